# baseline (device time: 172968 ns/iter reference)
import jax
import jax.numpy as jnp
from jax import lax
from jax.experimental import pallas as pl
from jax.experimental.pallas import tpu as pltpu

N_DEV = 4

FROM_L = 0
FROM_R = 1
OPP = 2


def kernel(x, router_W, route_idx, expert_W, shared_W):
    n_tok, d_model = x.shape
    e_per, _, d_h = expert_W.shape
    e_half = e_per // 2
    e_q = e_half // 2

    xb = x.astype(jnp.bfloat16)
    rwb = router_W.astype(jnp.bfloat16)

    def body(x_ref, rw_ref, idx_ref, ew_ref, sw_ref, out_ref,
             gather_ref, stage_ref, own_ref, p_ref, send_sems, recv_sems,
             load_sems):
        my = lax.axis_index("i")
        left = lax.rem(my + N_DEV - 1, N_DEV)
        right = lax.rem(my + 1, N_DEV)

        def rdma(src, dst, si, ri, dev):
            return pltpu.make_async_remote_copy(
                src_ref=src, dst_ref=dst,
                send_sem=send_sems.at[si], recv_sem=recv_sems.at[ri],
                device_id=(dev,), device_id_type=pl.DeviceIdType.MESH,
            )

        def stage_round(r):
            ld = pltpu.make_async_copy(
                ew_ref.at[pl.ds(r * e_q, e_q)], stage_ref,
                load_sems.at[r % 2])
            ld.start()
            ld.wait()
            for k in range(e_q):
                own_ref[r * e_q + k] = stage_ref[k].astype(jnp.bfloat16)

        stage_round(0)
        stage_round(1)

        barrier_sem = pltpu.get_barrier_semaphore()
        for nbr in (left, right):
            pl.semaphore_signal(
                barrier_sem, inc=1,
                device_id=(nbr,), device_id_type=pl.DeviceIdType.MESH,
            )
        pl.semaphore_wait(barrier_sem, 2)

        lo = pl.ds(0, e_half)
        hi = pl.ds(e_half, e_half)
        s1_lo_r = rdma(own_ref.at[lo], gather_ref.at[FROM_L, lo], 0, 0, right)
        s1_lo_l = rdma(own_ref.at[lo], gather_ref.at[FROM_R, lo], 1, 1, left)
        s1_hi_r = rdma(own_ref.at[hi], gather_ref.at[FROM_L, hi], 2, 2, right)
        s1_hi_l = rdma(own_ref.at[hi], gather_ref.at[FROM_R, hi], 3, 3, left)
        s1_lo_r.start()
        s1_lo_l.start()

        stage_round(2)
        stage_round(3)

        idx = idx_ref[...]
        scores = jnp.dot(x_ref[...], rw_ref[...],
                         preferred_element_type=jnp.float32)
        s_max = jnp.max(scores, axis=-1, keepdims=True)
        e_s = jnp.exp(scores - s_max)
        probs = e_s / jnp.sum(e_s, axis=-1, keepdims=True)
        eids = lax.broadcasted_iota(jnp.int32, scores.shape, 1)
        p_ref[...] = jnp.sum(jnp.where(eids == idx, probs, 0.0),
                             axis=-1, keepdims=True)

        s1_lo_r.wait_send()
        s1_hi_r.start()
        s1_lo_l.wait_send()
        s1_hi_l.start()

        TB = 512
        NB = n_tok // TB

        sw = sw_ref[...].astype(jnp.bfloat16)

        def shared_blk(b, c):
            sl = pl.ds(b * TB, TB)
            out_ref[sl, :] = jnp.dot(
                x_ref[sl, :], sw, preferred_element_type=jnp.float32
            ).astype(jnp.bfloat16)
            return c

        lax.fori_loop(0, NB, shared_blk, 0)

        def add_experts(origin, w_ref, js):
            def blk(b, c):
                sl = pl.ds(b * TB, TB)
                x_blk = x_ref[sl, :]
                idx_blk = idx_ref[sl, :]
                p_blk = p_ref[sl, :]
                acc = out_ref[sl, :].astype(jnp.float32)
                for j in js:
                    e_glob = origin * e_per + j
                    coeff = jnp.where(idx_blk == e_glob, p_blk,
                                      0.0).astype(jnp.bfloat16)
                    acc = acc + jnp.dot(x_blk * coeff, w_ref[j],
                                        preferred_element_type=jnp.float32)
                out_ref[sl, :] = acc.astype(jnp.bfloat16)
                return c

            lax.fori_loop(0, NB, blk, 0)

        def add_chunk(origin, w_ref, j0=0, nj=e_per):
            add_experts(origin, w_ref, range(j0, j0 + nj))

        add_chunk(my, own_ref)

        q = [pl.ds(k * e_q, e_q) for k in range(4)]
        f2l_q0 = rdma(gather_ref.at[FROM_R, q[0]], gather_ref.at[OPP, q[0]],
                      4, 4, left)
        f2l_q1 = rdma(gather_ref.at[FROM_R, q[1]], gather_ref.at[OPP, q[1]],
                      5, 5, left)
        f2r_q0 = rdma(gather_ref.at[FROM_L, q[2]], gather_ref.at[OPP, q[2]],
                      6, 6, right)
        f2r_q1 = rdma(gather_ref.at[FROM_L, q[3]], gather_ref.at[OPP, q[3]],
                      7, 7, right)

        s1_lo_r.wait_recv()
        s1_lo_l.wait_recv()
        add_chunk(left, gather_ref.at[FROM_L], 0, e_half)
        add_chunk(right, gather_ref.at[FROM_R], 0, e_half)

        s1_hi_l.wait_send()
        f2l_q0.start()
        s1_hi_r.wait_send()
        s1_hi_r.wait_recv()
        f2r_q0.start()

        s1_hi_l.wait_recv()
        add_chunk(left, gather_ref.at[FROM_L], e_half, e_half)
        add_chunk(right, gather_ref.at[FROM_R], e_half, e_half)

        f2l_q0.wait_send()
        f2l_q1.start()
        f2r_q0.wait_send()
        f2r_q1.start()

        opp = lax.rem(my + 2, N_DEV)
        f2l_q0.wait_recv()
        f2r_q0.wait_recv()
        add_experts(opp, gather_ref.at[OPP], (0, 1, 4, 5))
        f2l_q1.wait_recv()
        f2r_q1.wait_recv()
        add_experts(opp, gather_ref.at[OPP], (2, 3, 6, 7))

        f2l_q1.wait_send()
        f2r_q1.wait_send()

    return pl.pallas_call(
        body,
        out_shape=jax.ShapeDtypeStruct((n_tok, d_h), jnp.bfloat16),
        in_specs=[
            pl.BlockSpec(memory_space=pltpu.VMEM),
            pl.BlockSpec(memory_space=pltpu.VMEM),
            pl.BlockSpec(memory_space=pltpu.VMEM),
            pl.BlockSpec(memory_space=pltpu.MemorySpace.HBM),
            pl.BlockSpec(memory_space=pltpu.VMEM),
        ],
        out_specs=pl.BlockSpec(memory_space=pltpu.VMEM),
        scratch_shapes=[
            pltpu.VMEM((3, e_per, d_model, d_h), jnp.bfloat16),
            pltpu.VMEM((e_q, d_model, d_h), jnp.float32),
            pltpu.VMEM((e_per, d_model, d_h), jnp.bfloat16),
            pltpu.VMEM((n_tok, 1), jnp.float32),
            pltpu.SemaphoreType.DMA((8,)),
            pltpu.SemaphoreType.DMA((8,)),
            pltpu.SemaphoreType.DMA((2,)),
        ],
        compiler_params=pltpu.CompilerParams(
            collective_id=0, vmem_limit_bytes=53 * 1024 * 1024),
    )(xb, rwb, route_idx, expert_W, shared_W)


# device time: 134352 ns/iter; 1.2874x vs baseline; 1.2874x over previous
import jax
import jax.numpy as jnp
from jax import lax
from jax.experimental import pallas as pl
from jax.experimental.pallas import tpu as pltpu

N_DEV = 4

FROM_L = 0
FROM_R = 1
OPP = 2


def kernel(x, router_W, route_idx, expert_W, shared_W):
    n_tok, d_model = x.shape
    e_per, _, d_h = expert_W.shape
    e_half = e_per // 2
    e_q = e_half // 2

    xb = x.astype(jnp.bfloat16)
    rwb = router_W.astype(jnp.bfloat16)

    amax = jnp.max(jnp.abs(expert_W), axis=(1, 2), keepdims=True)
    step = amax / 127.0
    ew_q = jnp.round(expert_W / step).astype(jnp.int8)
    scales = jnp.zeros((1, 128), jnp.float32).at[0, :e_per].set(step[:, 0, 0])

    def body(x_ref, rw_ref, idx_ref, ew_ref, sw_ref, sc_ref, out_ref,
             gather_ref, stage_ref, own_ref, scs_ref, p_ref,
             send_sems, recv_sems, load_sems):
        my = lax.axis_index("i")
        left = lax.rem(my + N_DEV - 1, N_DEV)
        right = lax.rem(my + 1, N_DEV)

        def rdma(src, dst, si, ri, dev):
            return pltpu.make_async_remote_copy(
                src_ref=src, dst_ref=dst,
                send_sem=send_sems.at[si], recv_sem=recv_sems.at[ri],
                device_id=(dev,), device_id_type=pl.DeviceIdType.MESH,
            )

        def stage_round(r):
            ld = pltpu.make_async_copy(
                ew_ref.at[pl.ds(r * e_half, e_half)], stage_ref,
                load_sems.at[r % 2])
            ld.start()
            ld.wait()
            for k in range(e_half):
                own_ref[r * e_half + k] = stage_ref[k]

        stage_round(0)
        stage_round(1)

        barrier_sem = pltpu.get_barrier_semaphore()
        for nbr in (left, right):
            pl.semaphore_signal(
                barrier_sem, inc=1,
                device_id=(nbr,), device_id_type=pl.DeviceIdType.MESH,
            )
        pl.semaphore_wait(barrier_sem, 2)

        lo = pl.ds(0, e_half)
        hi = pl.ds(e_half, e_half)
        s1_lo_r = rdma(own_ref.at[lo], gather_ref.at[FROM_L, lo], 0, 0, right)
        s1_lo_l = rdma(own_ref.at[lo], gather_ref.at[FROM_R, lo], 1, 1, left)
        s1_hi_r = rdma(own_ref.at[hi], gather_ref.at[FROM_L, hi], 2, 2, right)
        s1_hi_l = rdma(own_ref.at[hi], gather_ref.at[FROM_R, hi], 3, 3, left)
        sc_r = rdma(sc_ref, scs_ref.at[FROM_L], 8, 8, right)
        sc_l = rdma(sc_ref, scs_ref.at[FROM_R], 9, 9, left)
        s1_lo_r.start()
        s1_lo_l.start()
        sc_r.start()
        sc_l.start()

        idx = idx_ref[...]
        scores = jnp.dot(x_ref[...], rw_ref[...],
                         preferred_element_type=jnp.float32)
        s_max = jnp.max(scores, axis=-1, keepdims=True)
        e_s = jnp.exp(scores - s_max)
        probs = e_s / jnp.sum(e_s, axis=-1, keepdims=True)
        eids = lax.broadcasted_iota(jnp.int32, scores.shape, 1)
        p_ref[...] = jnp.sum(jnp.where(eids == idx, probs, 0.0),
                             axis=-1, keepdims=True)

        s1_lo_r.wait_send()
        s1_hi_r.start()
        s1_lo_l.wait_send()
        s1_hi_l.start()

        TB = 512
        NB = n_tok // TB

        sw = sw_ref[...].astype(jnp.bfloat16)

        def shared_blk(b, c):
            sl = pl.ds(b * TB, TB)
            out_ref[sl, :] = jnp.dot(
                x_ref[sl, :], sw, preferred_element_type=jnp.float32
            ).astype(jnp.bfloat16)
            return c

        lax.fori_loop(0, NB, shared_blk, 0)

        def add_experts(origin, w_ref, js, s_at):
            def blk(b, c):
                sl = pl.ds(b * TB, TB)
                x_blk = x_ref[sl, :]
                idx_blk = idx_ref[sl, :]
                p_blk = p_ref[sl, :]
                acc = out_ref[sl, :].astype(jnp.float32)
                for j in js:
                    e_glob = origin * e_per + j
                    coeff = jnp.where(idx_blk == e_glob, p_blk * s_at(j),
                                      0.0).astype(jnp.bfloat16)
                    acc = acc + jnp.dot(
                        x_blk * coeff, w_ref[j].astype(jnp.bfloat16),
                        preferred_element_type=jnp.float32)
                out_ref[sl, :] = acc.astype(jnp.bfloat16)
                return c

            lax.fori_loop(0, NB, blk, 0)

        def add_chunk(origin, w_ref, s_at, j0=0, nj=e_per):
            add_experts(origin, w_ref, range(j0, j0 + nj), s_at)

        def slot_scale(slot):
            return lambda j: scs_ref[slot, 0, j]

        add_chunk(my, own_ref, lambda j: sc_ref[0, j])

        q = [pl.ds(k * e_q, e_q) for k in range(4)]
        f2l_q0 = rdma(gather_ref.at[FROM_R, q[0]], gather_ref.at[OPP, q[0]],
                      4, 4, left)
        f2l_q1 = rdma(gather_ref.at[FROM_R, q[1]], gather_ref.at[OPP, q[1]],
                      5, 5, left)
        f2r_q0 = rdma(gather_ref.at[FROM_L, q[2]], gather_ref.at[OPP, q[2]],
                      6, 6, right)
        f2r_q1 = rdma(gather_ref.at[FROM_L, q[3]], gather_ref.at[OPP, q[3]],
                      7, 7, right)
        sc_fwd = rdma(scs_ref.at[FROM_R], scs_ref.at[OPP], 10, 10, left)

        s1_lo_r.wait_recv()
        s1_lo_l.wait_recv()
        sc_r.wait_recv()
        sc_l.wait_recv()
        add_chunk(left, gather_ref.at[FROM_L], slot_scale(FROM_L), 0, e_half)
        add_chunk(right, gather_ref.at[FROM_R], slot_scale(FROM_R), 0, e_half)

        s1_hi_l.wait_send()
        f2l_q0.start()
        sc_fwd.start()
        s1_hi_r.wait_send()
        s1_hi_r.wait_recv()
        f2r_q0.start()

        s1_hi_l.wait_recv()
        add_chunk(left, gather_ref.at[FROM_L], slot_scale(FROM_L),
                  e_half, e_half)
        add_chunk(right, gather_ref.at[FROM_R], slot_scale(FROM_R),
                  e_half, e_half)

        f2l_q0.wait_send()
        f2l_q1.start()
        f2r_q0.wait_send()
        f2r_q1.start()

        opp = lax.rem(my + 2, N_DEV)
        sc_fwd.wait_recv()
        f2l_q0.wait_recv()
        f2r_q0.wait_recv()
        add_experts(opp, gather_ref.at[OPP], (0, 1, 4, 5), slot_scale(OPP))
        f2l_q1.wait_recv()
        f2r_q1.wait_recv()
        add_experts(opp, gather_ref.at[OPP], (2, 3, 6, 7), slot_scale(OPP))

        f2l_q1.wait_send()
        f2r_q1.wait_send()
        sc_fwd.wait_send()
        sc_r.wait_send()
        sc_l.wait_send()

    return pl.pallas_call(
        body,
        out_shape=jax.ShapeDtypeStruct((n_tok, d_h), jnp.bfloat16),
        in_specs=[
            pl.BlockSpec(memory_space=pltpu.VMEM),
            pl.BlockSpec(memory_space=pltpu.VMEM),
            pl.BlockSpec(memory_space=pltpu.VMEM),
            pl.BlockSpec(memory_space=pltpu.MemorySpace.HBM),
            pl.BlockSpec(memory_space=pltpu.VMEM),
            pl.BlockSpec(memory_space=pltpu.VMEM),
        ],
        out_specs=pl.BlockSpec(memory_space=pltpu.VMEM),
        scratch_shapes=[
            pltpu.VMEM((3, e_per, d_model, d_h), jnp.int8),
            pltpu.VMEM((e_half, d_model, d_h), jnp.int8),
            pltpu.VMEM((e_per, d_model, d_h), jnp.int8),
            pltpu.VMEM((3, 1, 128), jnp.float32),
            pltpu.VMEM((n_tok, 1), jnp.float32),
            pltpu.SemaphoreType.DMA((11,)),
            pltpu.SemaphoreType.DMA((11,)),
            pltpu.SemaphoreType.DMA((2,)),
        ],
        compiler_params=pltpu.CompilerParams(
            collective_id=0, vmem_limit_bytes=40 * 1024 * 1024),
    )(xb, rwb, route_idx, ew_q, shared_W, scales)


# device time: 129858 ns/iter; 1.3320x vs baseline; 1.0346x over previous
import jax
import jax.numpy as jnp
from jax import lax
from jax.experimental import pallas as pl
from jax.experimental.pallas import tpu as pltpu

N_DEV = 4

FROM_L = 0
FROM_R = 1
OPP = 2


def kernel(x, router_W, route_idx, expert_W, shared_W):
    n_tok, d_model = x.shape
    e_per, _, d_h = expert_W.shape
    e_half = e_per // 2
    e_q = e_half // 2

    xb = x.astype(jnp.bfloat16)
    rwb = router_W.astype(jnp.bfloat16)

    def body(x_ref, rw_ref, idx_ref, ew_ref, sw_ref, out_ref,
             gather_ref, stage_ref, own_ref, own_sc_ref, scs_ref, p_ref,
             send_sems, recv_sems, load_sems):
        my = lax.axis_index("i")
        left = lax.rem(my + N_DEV - 1, N_DEV)
        right = lax.rem(my + 1, N_DEV)

        def rdma(src, dst, si, ri, dev):
            return pltpu.make_async_remote_copy(
                src_ref=src, dst_ref=dst,
                send_sem=send_sems.at[si], recv_sem=recv_sems.at[ri],
                device_id=(dev,), device_id_type=pl.DeviceIdType.MESH,
            )

        def stage_round(r):
            ld = pltpu.make_async_copy(
                ew_ref.at[pl.ds(r * e_half, e_half)], stage_ref,
                load_sems.at[r % 2])
            ld.start()
            ld.wait()
            for k in range(e_half):
                w = stage_ref[k]
                m = jnp.max(jnp.abs(w))
                own_ref[r * e_half + k] = jnp.round(
                    w * (127.0 / m)).astype(jnp.int8)
                own_sc_ref[r * e_half + k, :] = jnp.full((128,), m / 127.0,
                                                         jnp.float32)

        stage_round(0)

        barrier_sem = pltpu.get_barrier_semaphore()
        for nbr in (left, right):
            pl.semaphore_signal(
                barrier_sem, inc=1,
                device_id=(nbr,), device_id_type=pl.DeviceIdType.MESH,
            )
        pl.semaphore_wait(barrier_sem, 2)

        lo = pl.ds(0, e_half)
        hi = pl.ds(e_half, e_half)
        s1_lo_r = rdma(own_ref.at[lo], gather_ref.at[FROM_L, lo], 0, 0, right)
        s1_lo_l = rdma(own_ref.at[lo], gather_ref.at[FROM_R, lo], 1, 1, left)
        s1_hi_r = rdma(own_ref.at[hi], gather_ref.at[FROM_L, hi], 2, 2, right)
        s1_hi_l = rdma(own_ref.at[hi], gather_ref.at[FROM_R, hi], 3, 3, left)
        sc_r = rdma(own_sc_ref, scs_ref.at[FROM_L], 8, 8, right)
        sc_l = rdma(own_sc_ref, scs_ref.at[FROM_R], 9, 9, left)
        s1_lo_r.start()
        s1_lo_l.start()

        stage_round(1)
        sc_r.start()
        sc_l.start()

        idx = idx_ref[...]
        scores = jnp.dot(x_ref[...], rw_ref[...],
                         preferred_element_type=jnp.float32)
        s_max = jnp.max(scores, axis=-1, keepdims=True)
        e_s = jnp.exp(scores - s_max)
        probs = e_s / jnp.sum(e_s, axis=-1, keepdims=True)
        eids = lax.broadcasted_iota(jnp.int32, scores.shape, 1)
        p_ref[...] = jnp.sum(jnp.where(eids == idx, probs, 0.0),
                             axis=-1, keepdims=True)

        s1_lo_r.wait_send()
        s1_hi_r.start()
        s1_lo_l.wait_send()
        s1_hi_l.start()

        TB = 512
        NB = n_tok // TB

        sw = sw_ref[...].astype(jnp.bfloat16)

        def shared_blk(b, c):
            sl = pl.ds(b * TB, TB)
            out_ref[sl, :] = jnp.dot(
                x_ref[sl, :], sw, preferred_element_type=jnp.float32
            ).astype(jnp.bfloat16)
            return c

        lax.fori_loop(0, NB, shared_blk, 0)

        def add_experts(origin, w_ref, js, s_at):
            ws = [w_ref[j].astype(jnp.bfloat16) for j in js]

            def blk(b, c):
                sl = pl.ds(b * TB, TB)
                x_blk = x_ref[sl, :]
                idx_blk = idx_ref[sl, :]
                p_blk = p_ref[sl, :]
                acc = out_ref[sl, :].astype(jnp.float32)
                for j, w_bf in zip(js, ws):
                    e_glob = origin * e_per + j
                    coeff = jnp.where(idx_blk == e_glob, p_blk * s_at(j),
                                      0.0).astype(jnp.bfloat16)
                    acc = acc + jnp.dot(
                        x_blk * coeff, w_bf,
                        preferred_element_type=jnp.float32)
                out_ref[sl, :] = acc.astype(jnp.bfloat16)
                return c

            lax.fori_loop(0, NB, blk, 0)

        def add_chunk(origin, w_ref, s_at, j0=0, nj=e_per):
            add_experts(origin, w_ref, range(j0, j0 + nj), s_at)

        def slot_scale(slot):
            return lambda j: scs_ref[slot, j, 0]

        add_chunk(my, own_ref, lambda j: own_sc_ref[j, 0])

        q = [pl.ds(k * e_q, e_q) for k in range(4)]
        f2l_q0 = rdma(gather_ref.at[FROM_R, q[0]], gather_ref.at[OPP, q[0]],
                      4, 4, left)
        f2l_q1 = rdma(gather_ref.at[FROM_R, q[1]], gather_ref.at[OPP, q[1]],
                      5, 5, left)
        f2r_q0 = rdma(gather_ref.at[FROM_L, q[2]], gather_ref.at[OPP, q[2]],
                      6, 6, right)
        f2r_q1 = rdma(gather_ref.at[FROM_L, q[3]], gather_ref.at[OPP, q[3]],
                      7, 7, right)
        sc_fwd = rdma(scs_ref.at[FROM_R], scs_ref.at[OPP], 10, 10, left)

        s1_lo_r.wait_recv()
        s1_lo_l.wait_recv()
        sc_r.wait_recv()
        sc_l.wait_recv()
        add_chunk(left, gather_ref.at[FROM_L], slot_scale(FROM_L), 0, e_half)
        add_chunk(right, gather_ref.at[FROM_R], slot_scale(FROM_R), 0, e_half)

        s1_hi_l.wait_send()
        f2l_q0.start()
        sc_fwd.start()
        s1_hi_r.wait_send()
        s1_hi_r.wait_recv()
        f2r_q0.start()

        s1_hi_l.wait_recv()
        add_chunk(left, gather_ref.at[FROM_L], slot_scale(FROM_L),
                  e_half, e_half)
        add_chunk(right, gather_ref.at[FROM_R], slot_scale(FROM_R),
                  e_half, e_half)

        f2l_q0.wait_send()
        f2l_q1.start()
        f2r_q0.wait_send()
        f2r_q1.start()

        opp = lax.rem(my + 2, N_DEV)
        sc_fwd.wait_recv()
        f2l_q0.wait_recv()
        f2r_q0.wait_recv()
        add_experts(opp, gather_ref.at[OPP], (0, 1, 4, 5), slot_scale(OPP))
        f2l_q1.wait_recv()
        f2r_q1.wait_recv()
        add_experts(opp, gather_ref.at[OPP], (2, 3, 6, 7), slot_scale(OPP))

        f2l_q1.wait_send()
        f2r_q1.wait_send()
        sc_fwd.wait_send()
        sc_r.wait_send()
        sc_l.wait_send()

    return pl.pallas_call(
        body,
        out_shape=jax.ShapeDtypeStruct((n_tok, d_h), jnp.bfloat16),
        in_specs=[
            pl.BlockSpec(memory_space=pltpu.VMEM),
            pl.BlockSpec(memory_space=pltpu.VMEM),
            pl.BlockSpec(memory_space=pltpu.VMEM),
            pl.BlockSpec(memory_space=pltpu.MemorySpace.HBM),
            pl.BlockSpec(memory_space=pltpu.VMEM),
        ],
        out_specs=pl.BlockSpec(memory_space=pltpu.VMEM),
        scratch_shapes=[
            pltpu.VMEM((3, e_per, d_model, d_h), jnp.int8),
            pltpu.VMEM((e_half, d_model, d_h), jnp.float32),
            pltpu.VMEM((e_per, d_model, d_h), jnp.int8),
            pltpu.VMEM((e_per, 128), jnp.float32),
            pltpu.VMEM((3, e_per, 128), jnp.float32),
            pltpu.VMEM((n_tok, 1), jnp.float32),
            pltpu.SemaphoreType.DMA((11,)),
            pltpu.SemaphoreType.DMA((11,)),
            pltpu.SemaphoreType.DMA((2,)),
        ],
        compiler_params=pltpu.CompilerParams(
            collective_id=0, vmem_limit_bytes=46 * 1024 * 1024),
    )(xb, rwb, route_idx, expert_W, shared_W)


# device time: 128879 ns/iter; 1.3421x vs baseline; 1.0076x over previous
import jax
import jax.numpy as jnp
from jax import lax
from jax.experimental import pallas as pl
from jax.experimental.pallas import tpu as pltpu

N_DEV = 4

FROM_L = 0
FROM_R = 1
OPP = 2


def kernel(x, router_W, route_idx, expert_W, shared_W):
    n_tok, d_model = x.shape
    e_per, _, d_h = expert_W.shape
    e_half = e_per // 2
    e_q = e_half // 2

    xb = x.astype(jnp.bfloat16)
    rwb = router_W.astype(jnp.bfloat16)

    def body(x_ref, rw_ref, idx_ref, ew_ref, sw_ref, out_ref,
             gather_ref, stage_ref, own_ref, own_sc_ref, scs_ref, p_ref,
             send_sems, recv_sems, load_sems):
        my = lax.axis_index("i")
        left = lax.rem(my + N_DEV - 1, N_DEV)
        right = lax.rem(my + 1, N_DEV)

        def rdma(src, dst, si, ri, dev):
            return pltpu.make_async_remote_copy(
                src_ref=src, dst_ref=dst,
                send_sem=send_sems.at[si], recv_sem=recv_sems.at[ri],
                device_id=(dev,), device_id_type=pl.DeviceIdType.MESH,
            )

        def stage_load(r):
            ld = pltpu.make_async_copy(
                ew_ref.at[pl.ds(r * e_half, e_half)], stage_ref,
                load_sems.at[r % 2])
            ld.start()
            return ld

        def stage_quant(r, ld):
            ld.wait()
            for k in range(e_half):
                w = stage_ref[k]
                m = jnp.max(jnp.abs(w))
                own_ref[r * e_half + k] = jnp.round(
                    w * (127.0 / m)).astype(jnp.int8)
                own_sc_ref[r * e_half + k, :] = jnp.full((128,), m / 127.0,
                                                         jnp.float32)

        ld0 = stage_load(0)

        barrier_sem = pltpu.get_barrier_semaphore()
        for nbr in (left, right):
            pl.semaphore_signal(
                barrier_sem, inc=1,
                device_id=(nbr,), device_id_type=pl.DeviceIdType.MESH,
            )
        pl.semaphore_wait(barrier_sem, 2)
        stage_quant(0, ld0)

        lo = pl.ds(0, e_half)
        hi = pl.ds(e_half, e_half)
        s1_lo_r = rdma(own_ref.at[lo], gather_ref.at[FROM_L, lo], 0, 0, right)
        s1_lo_l = rdma(own_ref.at[lo], gather_ref.at[FROM_R, lo], 1, 1, left)
        s1_hi_r = rdma(own_ref.at[hi], gather_ref.at[FROM_L, hi], 2, 2, right)
        s1_hi_l = rdma(own_ref.at[hi], gather_ref.at[FROM_R, hi], 3, 3, left)
        sc_r = rdma(own_sc_ref, scs_ref.at[FROM_L], 8, 8, right)
        sc_l = rdma(own_sc_ref, scs_ref.at[FROM_R], 9, 9, left)
        s1_lo_r.start()
        s1_lo_l.start()

        stage_quant(1, stage_load(1))
        sc_r.start()
        sc_l.start()

        idx = idx_ref[...]
        scores = jnp.dot(x_ref[...], rw_ref[...],
                         preferred_element_type=jnp.float32)
        s_max = jnp.max(scores, axis=-1, keepdims=True)
        e_s = jnp.exp(scores - s_max)
        probs = e_s / jnp.sum(e_s, axis=-1, keepdims=True)
        eids = lax.broadcasted_iota(jnp.int32, scores.shape, 1)
        p_ref[...] = jnp.sum(jnp.where(eids == idx, probs, 0.0),
                             axis=-1, keepdims=True)

        s1_lo_r.wait_send()
        s1_hi_r.start()
        s1_lo_l.wait_send()
        s1_hi_l.start()

        TB = 512
        NB = n_tok // TB

        sw = sw_ref[...].astype(jnp.bfloat16)

        def shared_blk(b, c):
            sl = pl.ds(b * TB, TB)
            out_ref[sl, :] = jnp.dot(
                x_ref[sl, :], sw, preferred_element_type=jnp.float32
            ).astype(jnp.bfloat16)
            return c

        lax.fori_loop(0, NB, shared_blk, 0)

        def add_experts(origin, w_ref, js, s_at):
            ws = [w_ref[j].astype(jnp.bfloat16) for j in js]

            def blk(b, c):
                sl = pl.ds(b * TB, TB)
                x_blk = x_ref[sl, :]
                idx_blk = idx_ref[sl, :]
                p_blk = p_ref[sl, :]
                acc = out_ref[sl, :].astype(jnp.float32)
                for j, w_bf in zip(js, ws):
                    e_glob = origin * e_per + j
                    coeff = jnp.where(idx_blk == e_glob, p_blk * s_at(j),
                                      0.0).astype(jnp.bfloat16)
                    acc = acc + jnp.dot(
                        x_blk * coeff, w_bf,
                        preferred_element_type=jnp.float32)
                out_ref[sl, :] = acc.astype(jnp.bfloat16)
                return c

            lax.fori_loop(0, NB, blk, 0)

        def add_chunk(origin, w_ref, s_at, j0=0, nj=e_per):
            add_experts(origin, w_ref, range(j0, j0 + nj), s_at)

        def slot_scale(slot):
            return lambda j: scs_ref[slot, j, 0]

        add_chunk(my, own_ref, lambda j: own_sc_ref[j, 0])

        q = [pl.ds(k * e_q, e_q) for k in range(4)]
        f2l_q0 = rdma(gather_ref.at[FROM_R, q[0]], gather_ref.at[OPP, q[0]],
                      4, 4, left)
        f2l_q1 = rdma(gather_ref.at[FROM_R, q[1]], gather_ref.at[OPP, q[1]],
                      5, 5, left)
        f2r_q0 = rdma(gather_ref.at[FROM_L, q[2]], gather_ref.at[OPP, q[2]],
                      6, 6, right)
        f2r_q1 = rdma(gather_ref.at[FROM_L, q[3]], gather_ref.at[OPP, q[3]],
                      7, 7, right)
        sc_fwd = rdma(scs_ref.at[FROM_R], scs_ref.at[OPP], 10, 10, left)

        s1_lo_r.wait_recv()
        s1_lo_l.wait_recv()
        sc_r.wait_recv()
        sc_l.wait_recv()
        add_chunk(left, gather_ref.at[FROM_L], slot_scale(FROM_L), 0, e_half)
        add_chunk(right, gather_ref.at[FROM_R], slot_scale(FROM_R), 0, e_half)

        s1_hi_l.wait_send()
        f2l_q0.start()
        sc_fwd.start()
        s1_hi_r.wait_send()
        s1_hi_r.wait_recv()
        f2r_q0.start()

        s1_hi_l.wait_recv()
        add_chunk(left, gather_ref.at[FROM_L], slot_scale(FROM_L),
                  e_half, e_half)
        add_chunk(right, gather_ref.at[FROM_R], slot_scale(FROM_R),
                  e_half, e_half)

        f2l_q0.wait_send()
        f2l_q1.start()
        f2r_q0.wait_send()
        f2r_q1.start()

        opp = lax.rem(my + 2, N_DEV)
        sc_fwd.wait_recv()
        f2l_q0.wait_recv()
        f2r_q0.wait_recv()
        add_experts(opp, gather_ref.at[OPP], (0, 1, 4, 5), slot_scale(OPP))
        f2l_q1.wait_recv()
        f2r_q1.wait_recv()
        add_experts(opp, gather_ref.at[OPP], (2, 3, 6, 7), slot_scale(OPP))

        f2l_q1.wait_send()
        f2r_q1.wait_send()
        sc_fwd.wait_send()
        sc_r.wait_send()
        sc_l.wait_send()

    return pl.pallas_call(
        body,
        out_shape=jax.ShapeDtypeStruct((n_tok, d_h), jnp.bfloat16),
        in_specs=[
            pl.BlockSpec(memory_space=pltpu.VMEM),
            pl.BlockSpec(memory_space=pltpu.VMEM),
            pl.BlockSpec(memory_space=pltpu.VMEM),
            pl.BlockSpec(memory_space=pltpu.MemorySpace.HBM),
            pl.BlockSpec(memory_space=pltpu.VMEM),
        ],
        out_specs=pl.BlockSpec(memory_space=pltpu.VMEM),
        scratch_shapes=[
            pltpu.VMEM((3, e_per, d_model, d_h), jnp.int8),
            pltpu.VMEM((e_half, d_model, d_h), jnp.float32),
            pltpu.VMEM((e_per, d_model, d_h), jnp.int8),
            pltpu.VMEM((e_per, 128), jnp.float32),
            pltpu.VMEM((3, e_per, 128), jnp.float32),
            pltpu.VMEM((n_tok, 1), jnp.float32),
            pltpu.SemaphoreType.DMA((11,)),
            pltpu.SemaphoreType.DMA((11,)),
            pltpu.SemaphoreType.DMA((2,)),
        ],
        compiler_params=pltpu.CompilerParams(
            collective_id=0, vmem_limit_bytes=46 * 1024 * 1024),
    )(xb, rwb, route_idx, expert_W, shared_W)
